# baseline (speedup 1.0000x reference)
import jax
import jax.numpy as jnp
from jax.experimental import pallas as pl
from jax.experimental.pallas import tpu as pltpu

INPUT_DIM = 768
HIDDEN_DIM = 16384
K = 64
N_TOKENS = 4096
BLOCK_A = 64
BLOCK_B = 256


def _key_to_float(k):
    b = k ^ ((k >> 31) & jnp.int32(0x7FFFFFFF))
    return jax.lax.bitcast_convert_type(b, jnp.float32)


def _enc_body(x_ref, mean_ref, we_hbm, be_ref, sp_ref, we_vmem, sem):
    @pl.when(pl.program_id(0) == 0)
    def _copy_weights():
        pltpu.make_async_copy(we_hbm, we_vmem, sem).start()
        pltpu.make_async_copy(we_hbm, we_vmem, sem).wait()

    xc = x_ref[...] - mean_ref[...]
    z = jax.lax.dot_general(
        xc, we_vmem[...], (((1,), (0,)), ((), ())),
        preferred_element_type=jnp.float32,
    )
    z = z + be_ref[...]

    ones = jnp.ones((HIDDEN_DIM, 1), jnp.float32)

    lo0 = jnp.full((BLOCK_A, 1), jnp.iinfo(jnp.int32).min, jnp.int32)
    hi0 = jnp.full((BLOCK_A, 1), jnp.iinfo(jnp.int32).max, jnp.int32)

    def step(_, lohi):
        lo, hi = lohi
        mid = (lo >> 1) + (hi >> 1) + (lo & hi & 1)
        fmid = _key_to_float(mid)
        ind = jnp.where(z > fmid, 1.0, 0.0)
        cnt = jax.lax.dot_general(
            ind, ones, (((1,), (0,)), ((), ())),
            preferred_element_type=jnp.float32,
        )
        pred = cnt >= K
        return jnp.where(pred, mid, lo), jnp.where(pred, hi, mid)

    lo, _ = jax.lax.fori_loop(0, 32, step, (lo0, hi0), unroll=True)
    thr = _key_to_float(lo)
    sp_ref[...] = jnp.where(z > thr, z, 0.0).astype(jnp.bfloat16)


def _dec_body(sp_ref, wd_hbm, bd_ref, mean_ref, o_ref, wd_vmem, sem):
    @pl.when(pl.program_id(0) == 0)
    def _copy_weights():
        pltpu.make_async_copy(wd_hbm, wd_vmem, sem).start()
        pltpu.make_async_copy(wd_hbm, wd_vmem, sem).wait()

    dec = jax.lax.dot_general(
        sp_ref[...], wd_vmem[...], (((1,), (0,)), ((), ())),
        preferred_element_type=jnp.float32,
    )
    o_ref[...] = dec + bd_ref[...] + mean_ref[...]


@jax.jit
def kernel(x, W_enc, b_enc, W_dec, b_dec, mean):
    mean2 = mean.reshape(1, INPUT_DIM)
    sparse = pl.pallas_call(
        _enc_body,
        grid=(N_TOKENS // BLOCK_A,),
        in_specs=[
            pl.BlockSpec((BLOCK_A, INPUT_DIM), lambda i: (i, 0)),
            pl.BlockSpec((1, INPUT_DIM), lambda i: (0, 0)),
            pl.BlockSpec(memory_space=pl.ANY),
            pl.BlockSpec((1, HIDDEN_DIM), lambda i: (0, 0)),
        ],
        out_specs=pl.BlockSpec((BLOCK_A, HIDDEN_DIM), lambda i: (i, 0)),
        out_shape=jax.ShapeDtypeStruct((N_TOKENS, HIDDEN_DIM), jnp.bfloat16),
        scratch_shapes=[
            pltpu.VMEM((INPUT_DIM, HIDDEN_DIM), jnp.float32),
            pltpu.SemaphoreType.DMA,
        ],
    )(x, mean2, W_enc, b_enc.reshape(1, HIDDEN_DIM))

    return pl.pallas_call(
        _dec_body,
        grid=(N_TOKENS // BLOCK_B,),
        in_specs=[
            pl.BlockSpec((BLOCK_B, HIDDEN_DIM), lambda i: (i, 0)),
            pl.BlockSpec(memory_space=pl.ANY),
            pl.BlockSpec((1, INPUT_DIM), lambda i: (0, 0)),
            pl.BlockSpec((1, INPUT_DIM), lambda i: (0, 0)),
        ],
        out_specs=pl.BlockSpec((BLOCK_B, INPUT_DIM), lambda i: (i, 0)),
        out_shape=jax.ShapeDtypeStruct((N_TOKENS, INPUT_DIM), jnp.float32),
        scratch_shapes=[
            pltpu.VMEM((HIDDEN_DIM, INPUT_DIM), jnp.bfloat16),
            pltpu.SemaphoreType.DMA,
        ],
    )(sparse, W_dec.astype(jnp.bfloat16), b_dec.reshape(1, INPUT_DIM), mean2)


# speedup vs baseline: 1.3944x; 1.3944x over previous
import jax
import jax.numpy as jnp
from jax.experimental import pallas as pl
from jax.experimental.pallas import tpu as pltpu

INPUT_DIM = 768
HIDDEN_DIM = 16384
K = 64
N_TOKENS = 4096
BLOCK_A = 64
BLOCK_B = 256


def _key_to_float(k):
    b = k ^ ((k >> 31) & jnp.int32(0x7FFFFFFF))
    return jax.lax.bitcast_convert_type(b, jnp.float32)


def _enc_body(x_ref, mean_ref, we_hbm, be_ref, sp_ref, we_vmem, sem):
    @pl.when(pl.program_id(0) == 0)
    def _copy_weights():
        pltpu.make_async_copy(we_hbm, we_vmem, sem).start()
        pltpu.make_async_copy(we_hbm, we_vmem, sem).wait()

    xc = x_ref[...] - mean_ref[...]
    z = jax.lax.dot_general(
        xc, we_vmem[...], (((1,), (0,)), ((), ())),
        preferred_element_type=jnp.float32,
    )
    z = z + be_ref[...]

    lo0 = jnp.full((BLOCK_A, 1), jnp.iinfo(jnp.int32).min, jnp.int32)
    hi0 = jnp.full((BLOCK_A, 1), jnp.iinfo(jnp.int32).max, jnp.int32)

    def step(_, lohi):
        lo, hi = lohi
        mid = (lo >> 1) + (hi >> 1) + (lo & hi & 1)
        fmid = _key_to_float(mid)
        cnt = jnp.sum((z > fmid).astype(jnp.float32), axis=1, keepdims=True)
        pred = cnt >= K
        return jnp.where(pred, mid, lo), jnp.where(pred, hi, mid)

    lo, _ = jax.lax.fori_loop(0, 32, step, (lo0, hi0), unroll=True)
    thr = _key_to_float(lo)
    sp_ref[...] = jnp.where(z > thr, z, 0.0).astype(jnp.bfloat16)


def _dec_body(sp_ref, wd_hbm, bd_ref, mean_ref, o_ref, wd_vmem, sem):
    @pl.when(pl.program_id(0) == 0)
    def _copy_weights():
        pltpu.make_async_copy(wd_hbm, wd_vmem, sem).start()
        pltpu.make_async_copy(wd_hbm, wd_vmem, sem).wait()

    dec = jax.lax.dot_general(
        sp_ref[...], wd_vmem[...], (((1,), (0,)), ((), ())),
        preferred_element_type=jnp.float32,
    )
    o_ref[...] = dec + bd_ref[...] + mean_ref[...]


@jax.jit
def kernel(x, W_enc, b_enc, W_dec, b_dec, mean):
    mean2 = mean.reshape(1, INPUT_DIM)
    sparse = pl.pallas_call(
        _enc_body,
        grid=(N_TOKENS // BLOCK_A,),
        in_specs=[
            pl.BlockSpec((BLOCK_A, INPUT_DIM), lambda i: (i, 0)),
            pl.BlockSpec((1, INPUT_DIM), lambda i: (0, 0)),
            pl.BlockSpec(memory_space=pl.ANY),
            pl.BlockSpec((1, HIDDEN_DIM), lambda i: (0, 0)),
        ],
        out_specs=pl.BlockSpec((BLOCK_A, HIDDEN_DIM), lambda i: (i, 0)),
        out_shape=jax.ShapeDtypeStruct((N_TOKENS, HIDDEN_DIM), jnp.bfloat16),
        scratch_shapes=[
            pltpu.VMEM((INPUT_DIM, HIDDEN_DIM), jnp.float32),
            pltpu.SemaphoreType.DMA,
        ],
    )(x, mean2, W_enc, b_enc.reshape(1, HIDDEN_DIM))

    return pl.pallas_call(
        _dec_body,
        grid=(N_TOKENS // BLOCK_B,),
        in_specs=[
            pl.BlockSpec((BLOCK_B, HIDDEN_DIM), lambda i: (i, 0)),
            pl.BlockSpec(memory_space=pl.ANY),
            pl.BlockSpec((1, INPUT_DIM), lambda i: (0, 0)),
            pl.BlockSpec((1, INPUT_DIM), lambda i: (0, 0)),
        ],
        out_specs=pl.BlockSpec((BLOCK_B, INPUT_DIM), lambda i: (i, 0)),
        out_shape=jax.ShapeDtypeStruct((N_TOKENS, INPUT_DIM), jnp.float32),
        scratch_shapes=[
            pltpu.VMEM((HIDDEN_DIM, INPUT_DIM), jnp.bfloat16),
            pltpu.SemaphoreType.DMA,
        ],
    )(sparse, W_dec.astype(jnp.bfloat16), b_dec.reshape(1, INPUT_DIM), mean2)
